# baseline (speedup 1.0000x reference)
import functools

import jax
import jax.numpy as jnp
from jax import lax
from jax.experimental import pallas as pl
from jax.experimental.pallas import tpu as pltpu
from jax.experimental.pallas import tpu_sc as plsc

D = 64
NC = 2
NS = 16
NW = NC * NS
NBUF = 4


def _make_sc_gather(S: int, T: int):
    rows_per_w = S // NW
    assert S % NW == 0 and rows_per_w % NBUF == 0 and rows_per_w >= 2 * NBUF
    mesh = plsc.VectorSubcoreMesh(core_axis_name="c", subcore_axis_name="s")

    @functools.partial(
        pl.kernel,
        out_type=jax.ShapeDtypeStruct((S, T, D), jnp.float32),
        mesh=mesh,
        scratch_types=(
            [pltpu.VMEM((T,), jnp.int32) for _ in range(NBUF)]
            + [pltpu.VMEM((T, D), jnp.float32) for _ in range(NBUF)]
            + [pltpu.SemaphoreType.DMA] * (3 * NBUF)
        ),
        compiler_params=pltpu.CompilerParams(use_tc_tiling_on_sc=False),
    )
    def sc_gather(table_hbm, tok_hbm, out_hbm, *bufs):
        idxs = bufs[0:NBUF]
        rows = bufs[NBUF:2 * NBUF]
        si = bufs[2 * NBUF:3 * NBUF]
        sg = bufs[3 * NBUF:4 * NBUF]
        ss = bufs[4 * NBUF:5 * NBUF]
        wid = lax.axis_index("s") * NC + lax.axis_index("c")
        wrow = wid * rows_per_w

        def idx_desc(r, b):
            return pltpu.make_async_copy(tok_hbm.at[wrow + r], idxs[b], si[b])

        def gather_desc(b):
            return pltpu.make_async_copy(table_hbm.at[idxs[b]], rows[b], sg[b])

        def scatter_desc(r, b):
            return pltpu.make_async_copy(rows[b], out_hbm.at[wrow + r], ss[b])

        def step(r, b, *, scat_wait, idx_next):
            idx_desc(r, b).wait()
            if scat_wait:
                scatter_desc(r - NBUF, b).wait()
            gather_desc(b).start()
            b2 = (b - 2) % NBUF
            gather_desc(b2).wait()
            scatter_desc(r - 2, b2).start()
            idx_desc(idx_next, (b + 2) % NBUF).start()

        for b in range(NBUF):
            idx_desc(b, b).start()
        idx_desc(0, 0).wait()
        gather_desc(0).start()
        idx_desc(1, 1).wait()
        gather_desc(1).start()
        for r in (2, 3):
            step(r, r, scat_wait=False, idx_next=r + 2)

        @pl.loop(NBUF, rows_per_w, step=NBUF)
        def _(r0):
            for b in range(NBUF):
                r = r0 + b
                r_next = jnp.minimum(r + 2, rows_per_w - 1)
                step(r, b, scat_wait=True, idx_next=r_next)

        n = rows_per_w
        gather_desc((n - 2) % NBUF).wait()
        scatter_desc(n - 2, (n - 2) % NBUF).start()
        gather_desc((n - 1) % NBUF).wait()
        scatter_desc(n - 1, (n - 1) % NBUF).start()
        idx_desc(n - 1, 0).wait()
        idx_desc(n - 1, 1).wait()
        for b in range(NBUF):
            scatter_desc(n - NBUF + b, (n - NBUF + b) % NBUF).wait()

    return sc_gather


def kernel(tokens, embed_weight):
    S, T = tokens.shape
    return _make_sc_gather(S, T)(embed_weight, tokens)


# speedup vs baseline: 1.2157x; 1.2157x over previous
import functools

import jax
import jax.numpy as jnp
from jax import lax
from jax.experimental import pallas as pl
from jax.experimental.pallas import tpu as pltpu
from jax.experimental.pallas import tpu_sc as plsc

D = 64
DP = 128
NC = 2
NS = 16
NW = NC * NS
CI = 1024
CG = 256
NSUB = CI // CG


def _make_sc_gather(B: int):
    b_per_w = B // NW
    nchunk = b_per_w // CI
    assert B % NW == 0 and b_per_w % CI == 0 and nchunk >= 3 and nchunk % 2 == 1
    mesh = plsc.VectorSubcoreMesh(core_axis_name="c", subcore_axis_name="s")

    @functools.partial(
        pl.kernel,
        out_type=jax.ShapeDtypeStruct((B, DP), jnp.float32),
        mesh=mesh,
        scratch_types=(
            [pltpu.VMEM((CI,), jnp.int32) for _ in range(2)]
            + [pltpu.VMEM((CG, DP), jnp.float32) for _ in range(2)]
            + [pltpu.SemaphoreType.DMA] * 6
        ),
        compiler_params=pltpu.CompilerParams(use_tc_tiling_on_sc=True),
    )
    def sc_gather(table_hbm, tok_hbm, out_hbm, idx0, idx1, rows0, rows1,
                  si0, si1, sg0, sg1, ss0, ss1):
        idxs = (idx0, idx1)
        rows = (rows0, rows1)
        si = (si0, si1)
        sg = (sg0, sg1)
        ss = (ss0, ss1)
        wid = lax.axis_index("s") * NC + lax.axis_index("c")
        wbase = pl.multiple_of(wid * b_per_w, CI)

        def idx_desc(c, p):
            base = pl.multiple_of(wbase + c * CI, CI)
            return pltpu.make_async_copy(
                tok_hbm.at[pl.ds(base, CI)], idxs[p], si[p])

        def gather_desc(k, p):
            return pltpu.make_async_copy(
                table_hbm.at[idxs[p].at[pl.ds(k * CG, CG)]],
                rows[k % 2], sg[k % 2])

        def scatter_desc(c, k):
            base = pl.multiple_of(wbase + c * CI + k * CG, 8)
            return pltpu.make_async_copy(
                rows[k % 2], out_hbm.at[pl.ds(base, CG)], ss[k % 2])

        def chunk_body(c, p, first):
            if first:
                idx_desc(c, p).wait()
                gather_desc(0, p).start()
                idx_desc(c + 1, 1 - p).start()
                for k in (1, 2, 3):
                    gather_desc(k - 1, p).wait()
                    scatter_desc(c, k - 1).start()
                    if k >= 2:
                        scatter_desc(c, k - 2).wait()
                    gather_desc(k, p).start()
            else:
                gather_desc(3, 1 - p).wait()
                scatter_desc(c - 1, 3).start()
                c_next = jnp.minimum(c + 1, nchunk - 1)
                idx_desc(c_next, 1 - p).start()
                idx_desc(c, p).wait()
                scatter_desc(c - 1, 2).wait()
                gather_desc(0, p).start()
                for k in (1, 2, 3):
                    gather_desc(k - 1, p).wait()
                    scatter_desc(c, k - 1).start()
                    cw, kw = (c, k - 2) if k >= 2 else (c - 1, 3)
                    scatter_desc(cw, kw).wait()
                    gather_desc(k, p).start()

        idx_desc(0, 0).start()
        chunk_body(0, 0, True)

        @pl.loop(1, nchunk, step=2)
        def _(c0):
            chunk_body(c0, 1, False)
            chunk_body(c0 + 1, 0, False)

        gather_desc(3, (nchunk - 1) % 2).wait()
        scatter_desc(nchunk - 1, 3).start()
        scatter_desc(nchunk - 1, 2).wait()
        scatter_desc(nchunk - 1, 3).wait()
        idx_desc(nchunk - 1, nchunk % 2).wait()

    return sc_gather


def kernel(tokens, embed_weight):
    S, T = tokens.shape
    B = S * T
    table128 = jnp.pad(embed_weight, ((0, 0), (0, DP - D)))
    flat = tokens.reshape(B)
    out = _make_sc_gather(B)(table128, flat)
    return out[:, :D].reshape(S, T, D)
